# Initial kernel scaffold; baseline (speedup 1.0000x reference)
import jax
import jax.numpy as jnp
from jax.experimental import pallas as pl


def kernel(atom_types, extra_table, W_onehot, electron_config, W_config, W1):
    raise NotImplementedError("write your pallas kernel here")


# speedup vs baseline: 3.9883x; 3.9883x over previous
import functools

import jax
import jax.numpy as jnp
from jax import lax
from jax.experimental import pallas as pl
from jax.experimental.pallas import tpu as pltpu
from jax.experimental.pallas import tpu_sc as plsc

_NUM_TYPES = 87
_F = 128
_N = 100000
_CH = 128


def _fuse_body(extra_ref, onehot_ref, econf_ref, wconf_ref, w1_ref, out_ref):
    cfg = lax.dot_general(
        econf_ref[...], wconf_ref[...], (((1,), (1,)), ((), ())),
        preferred_element_type=jnp.float32)
    cat = jnp.concatenate([extra_ref[...], onehot_ref[...], cfg], axis=1)
    out_ref[...] = lax.dot_general(
        cat, w1_ref[...], (((1,), (1,)), ((), ())),
        preferred_element_type=jnp.float32)


def _fused_table(extra, onehot, econf, wconf, w1):
    return pl.pallas_call(
        _fuse_body,
        out_shape=jax.ShapeDtypeStruct((_NUM_TYPES, _F), jnp.float32),
    )(extra, onehot, econf, wconf, w1)


@functools.cache
def _make_gather():
    info = plsc.get_sparse_core_info()
    nc, ns = info.num_cores, info.num_subcores
    nw = nc * ns
    b_per_w = ((_N + nw - 1) // nw + _CH - 1) // _CH * _CH
    max_chunks = b_per_w // _CH
    tail_r0 = (_N // _CH) * _CH
    tail_n = _N - tail_r0

    mesh = plsc.VectorSubcoreMesh(core_axis_name="c", subcore_axis_name="s")

    @functools.partial(
        pl.kernel,
        out_type=jax.ShapeDtypeStruct((_N, _F), jnp.float32),
        mesh=mesh,
        scratch_types=[
            pltpu.VMEM((_CH,), jnp.int32),
            pltpu.VMEM((_CH, _F), jnp.float32),
            pltpu.SemaphoreType.DMA,
        ],
    )
    def gather_k(idx_hbm, table_hbm, out_hbm, idx_v, rows_v, sem):
        wid = lax.axis_index("s") * nc + lax.axis_index("c")
        start = pl.multiple_of(wid * b_per_w, _CH)
        n_full = jnp.minimum(max_chunks, lax.div(_N - start, _CH))

        def chunk(i, carry):
            r0 = pl.multiple_of(start + i * _CH, _CH)
            pltpu.sync_copy(idx_hbm.at[pl.ds(r0, _CH)], idx_v)
            pltpu.async_copy(table_hbm.at[idx_v], rows_v, sem).wait()
            pltpu.sync_copy(rows_v, out_hbm.at[pl.ds(r0, _CH)])
            return carry

        lax.fori_loop(0, n_full, chunk, 0)

        @pl.when(start + n_full * _CH == tail_r0)
        def _():
            idx_t = idx_v.at[pl.ds(0, tail_n)]
            rows_t = rows_v.at[pl.ds(0, tail_n)]
            pltpu.sync_copy(idx_hbm.at[pl.ds(tail_r0, tail_n)], idx_t)
            pltpu.async_copy(table_hbm.at[idx_t], rows_t, sem).wait()
            pltpu.sync_copy(rows_t, out_hbm.at[pl.ds(tail_r0, tail_n)])

    return gather_k


def kernel(atom_types, extra_table, W_onehot, electron_config, W_config, W1):
    table = _fused_table(extra_table, W_onehot, electron_config, W_config, W1)
    out = _make_gather()(atom_types.astype(jnp.int32), table)
    return out, out
